# Initial kernel scaffold; baseline (speedup 1.0000x reference)
import jax
import jax.numpy as jnp
from jax.experimental import pallas as pl


def kernel(x, src0, src1, W_ih0, W_hh0, b_ih0, b_hh0, Ws0, Wn0, bc0, W_ih1, W_hh1, b_ih1, b_hh1, Ws1, Wn1, bc1, ln1_g, ln1_b, ln3_g, ln3_b):
    raise NotImplementedError("write your pallas kernel here")


# speedup vs baseline: 4.9130x; 4.9130x over previous
import functools

import jax
import jax.numpy as jnp
from jax import lax
from jax.experimental import pallas as pl
from jax.experimental.pallas import tpu as pltpu
from jax.experimental.pallas import tpu_sc as plsc

_SC_CORES = 2
_SC_SUBCORES = 16
_NW = _SC_CORES * _SC_SUBCORES


def _sc_gather(x, idx_all):
    tot = idx_all.shape[0]
    d = x.shape[1]
    rows_w = tot // _NW
    assert rows_w * _NW == tot
    chunk = 128
    nfull = rows_w // chunk
    tail = rows_w - nfull * chunk
    chunks = [(j * chunk, chunk) for j in range(nfull)]
    if tail:
        chunks.append((nfull * chunk, tail))
    nch = len(chunks)

    mesh = plsc.VectorSubcoreMesh(core_axis_name="c", subcore_axis_name="s")

    @functools.partial(
        pl.kernel,
        out_type=jax.ShapeDtypeStruct((tot, d), x.dtype),
        mesh=mesh,
        scratch_types=[
            pltpu.VMEM((rows_w,), jnp.int32),
            pltpu.VMEM((chunk, d), x.dtype),
            pltpu.VMEM((chunk, d), x.dtype),
            pltpu.SemaphoreType.DMA,
            pltpu.SemaphoreType.DMA,
            pltpu.SemaphoreType.DMA,
            pltpu.SemaphoreType.DMA,
        ],
    )
    def gk(x_hbm, idx_hbm, out_hbm, idx_v, buf0, buf1, g0, g1, s0, s1):
        wid = lax.axis_index("s") * _SC_CORES + lax.axis_index("c")
        base = wid * rows_w
        pltpu.sync_copy(idx_hbm.at[pl.ds(base, rows_w)], idx_v)
        bufs = (buf0, buf1)
        gsems = (g0, g1)
        ssems = (s0, s1)
        gh = [None] * nch
        sh = [None] * nch
        off0, c0 = chunks[0]
        gh[0] = pltpu.async_copy(
            x_hbm.at[idx_v.at[pl.ds(off0, c0)]], bufs[0].at[pl.ds(0, c0)], gsems[0]
        )
        for j in range(nch):
            b = j & 1
            gh[j].wait()
            if j + 1 < nch:
                if j >= 1:
                    sh[j - 1].wait()
                offn, cn = chunks[j + 1]
                gh[j + 1] = pltpu.async_copy(
                    x_hbm.at[idx_v.at[pl.ds(offn, cn)]],
                    bufs[1 - b].at[pl.ds(0, cn)],
                    gsems[1 - b],
                )
            offj, cj = chunks[j]
            sh[j] = pltpu.async_copy(
                bufs[b].at[pl.ds(0, cj)],
                out_hbm.at[pl.ds(base + offj, cj)],
                ssems[b],
            )
        sh[nch - 1].wait()
        if nch >= 2:
            sh[nch - 2].wait()

    return gk(x, idx_all)


def _leaky(z):
    return jnp.where(z >= 0, z, 0.01 * z)


def _layer_norm(z, g, b):
    mu = jnp.mean(z, axis=-1, keepdims=True)
    var = jnp.mean((z - mu) * (z - mu), axis=-1, keepdims=True)
    return (z - mu) / jnp.sqrt(var + 1e-5) * g + b


def _lstm_body(neigh_ref, x_ref, a0_ref, a1_ref, b0_ref, b1_ref, p_ref, bc_ref,
               g1_ref, bb1_ref, g3_ref, bb3_ref, out_ref, h0, c0, h1, c1,
               *, deg, d):
    t = pl.program_id(1)

    @pl.when(t == 0)
    def _():
        h0[...] = jnp.zeros_like(h0)
        c0[...] = jnp.zeros_like(c0)
        h1[...] = jnp.zeros_like(h1)
        c1[...] = jnp.zeros_like(c1)

    for r, (h, c, a_ref, b_ref) in enumerate(
        ((h0, c0, a0_ref, b0_ref), (h1, c1, a1_ref, b1_ref))
    ):
        xt = neigh_ref[r, 0]
        xh = jnp.concatenate([xt, h[...]], axis=1)
        gates = jnp.dot(xh, a_ref[...], preferred_element_type=jnp.float32)
        gates = gates + b_ref[...]
        ii = gates[:, 0 * d:1 * d]
        ff = gates[:, 1 * d:2 * d]
        gg = gates[:, 2 * d:3 * d]
        oo = gates[:, 3 * d:4 * d]
        cn = jax.nn.sigmoid(ff) * c[...] + jax.nn.sigmoid(ii) * jnp.tanh(gg)
        c[...] = cn
        h[...] = jax.nn.sigmoid(oo) * jnp.tanh(cn)

    @pl.when(t == deg - 1)
    def _():
        xb = x_ref[...]
        hcat = jnp.concatenate([xb, h0[...], h1[...]], axis=1)
        y = jnp.dot(hcat, p_ref[...], preferred_element_type=jnp.float32)
        y = y + bc_ref[...]
        y = _leaky(_layer_norm(y, g1_ref[...], bb1_ref[...]))
        y = xb + y
        y = _leaky(_layer_norm(y, g3_ref[...], bb3_ref[...]))
        out_ref[...] = y


def _tc_lstm(neigh, x, a0, a1, b0, b1, p, bc, g1, bb1, g3, bb3, bn):
    n, d = x.shape
    deg = neigh.shape[1]
    grid = (n // bn, deg)
    full = lambda i, t: (0, 0)
    return pl.pallas_call(
        functools.partial(_lstm_body, deg=deg, d=d),
        grid=grid,
        in_specs=[
            pl.BlockSpec((2, 1, bn, d), lambda i, t: (0, t, i, 0)),
            pl.BlockSpec((bn, d), lambda i, t: (i, 0)),
            pl.BlockSpec((2 * d, 4 * d), full),
            pl.BlockSpec((2 * d, 4 * d), full),
            pl.BlockSpec((1, 4 * d), full),
            pl.BlockSpec((1, 4 * d), full),
            pl.BlockSpec((3 * d, d), full),
            pl.BlockSpec((1, d), full),
            pl.BlockSpec((1, d), full),
            pl.BlockSpec((1, d), full),
            pl.BlockSpec((1, d), full),
            pl.BlockSpec((1, d), full),
        ],
        out_specs=pl.BlockSpec((bn, d), lambda i, t: (i, 0)),
        out_shape=jax.ShapeDtypeStruct((n, d), jnp.float32),
        scratch_shapes=[pltpu.VMEM((bn, d), jnp.float32) for _ in range(4)],
        compiler_params=pltpu.CompilerParams(
            dimension_semantics=("arbitrary", "arbitrary"),
        ),
    )(neigh, x, a0, a1, b0, b1, p, bc, g1, bb1, g3, bb3)


def kernel(x, src0, src1, W_ih0, W_hh0, b_ih0, b_hh0, Ws0, Wn0, bc0,
           W_ih1, W_hh1, b_ih1, b_hh1, Ws1, Wn1, bc1,
           ln1_g, ln1_b, ln3_g, ln3_b):
    n, d = x.shape
    deg = src0.shape[0] // n

    idx0 = src0.astype(jnp.int32).reshape(n, deg).T
    idx1 = src1.astype(jnp.int32).reshape(n, deg).T
    idx_all = jnp.concatenate([idx0.reshape(-1), idx1.reshape(-1)])

    neigh = _sc_gather(x, idx_all).reshape(2, deg, n, d)

    a0 = jnp.concatenate([W_ih0.T, W_hh0.T], axis=0)
    a1 = jnp.concatenate([W_ih1.T, W_hh1.T], axis=0)
    b0 = (b_ih0 + b_hh0).reshape(1, 4 * d)
    b1 = (b_ih1 + b_hh1).reshape(1, 4 * d)
    p = jnp.concatenate([(Ws0 + Ws1).T, Wn0.T, Wn1.T], axis=0)
    bc = (bc0 + bc1).reshape(1, d)

    bn = 1000
    return _tc_lstm(neigh, x, a0, a1, b0, b1, p, bc,
                    ln1_g.reshape(1, d), ln1_b.reshape(1, d),
                    ln3_g.reshape(1, d), ln3_b.reshape(1, d), bn)


# speedup vs baseline: 4.9198x; 1.0014x over previous
import functools

import jax
import jax.numpy as jnp
from jax import lax
from jax.experimental import pallas as pl
from jax.experimental.pallas import tpu as pltpu
from jax.experimental.pallas import tpu_sc as plsc

_SC_CORES = 2
_SC_SUBCORES = 16
_NW = _SC_CORES * _SC_SUBCORES


def _sc_gather(x, idx_all):
    tot = idx_all.shape[0]
    d = x.shape[1]
    rows_w = tot // _NW
    assert rows_w * _NW == tot
    chunk = 128
    nfull = rows_w // chunk
    tail = rows_w - nfull * chunk
    chunks = [(j * chunk, chunk) for j in range(nfull)]
    if tail:
        chunks.append((nfull * chunk, tail))
    nch = len(chunks)

    mesh = plsc.VectorSubcoreMesh(core_axis_name="c", subcore_axis_name="s")

    @functools.partial(
        pl.kernel,
        out_type=jax.ShapeDtypeStruct((tot, d), x.dtype),
        mesh=mesh,
        scratch_types=[
            pltpu.VMEM((rows_w,), jnp.int32),
            pltpu.VMEM((chunk, d), x.dtype),
            pltpu.VMEM((chunk, d), x.dtype),
            pltpu.SemaphoreType.DMA,
            pltpu.SemaphoreType.DMA,
            pltpu.SemaphoreType.DMA,
            pltpu.SemaphoreType.DMA,
        ],
    )
    def gk(x_hbm, idx_hbm, out_hbm, idx_v, buf0, buf1, g0, g1, s0, s1):
        wid = lax.axis_index("s") * _SC_CORES + lax.axis_index("c")
        base = wid * rows_w
        pltpu.sync_copy(idx_hbm.at[pl.ds(base, rows_w)], idx_v)
        bufs = (buf0, buf1)
        gsems = (g0, g1)
        ssems = (s0, s1)
        gh = [None] * nch
        sh = [None] * nch
        off0, c0 = chunks[0]
        gh[0] = pltpu.async_copy(
            x_hbm.at[idx_v.at[pl.ds(off0, c0)]], bufs[0].at[pl.ds(0, c0)], gsems[0]
        )
        for j in range(nch):
            b = j & 1
            gh[j].wait()
            if j + 1 < nch:
                if j >= 1:
                    sh[j - 1].wait()
                offn, cn = chunks[j + 1]
                gh[j + 1] = pltpu.async_copy(
                    x_hbm.at[idx_v.at[pl.ds(offn, cn)]],
                    bufs[1 - b].at[pl.ds(0, cn)],
                    gsems[1 - b],
                )
            offj, cj = chunks[j]
            sh[j] = pltpu.async_copy(
                bufs[b].at[pl.ds(0, cj)],
                out_hbm.at[pl.ds(base + offj, cj)],
                ssems[b],
            )
        sh[nch - 1].wait()
        if nch >= 2:
            sh[nch - 2].wait()

    return gk(x, idx_all)


def _leaky(z):
    return jnp.where(z >= 0, z, 0.01 * z)


def _layer_norm(z, g, b):
    mu = jnp.mean(z, axis=-1, keepdims=True)
    var = jnp.mean((z - mu) * (z - mu), axis=-1, keepdims=True)
    return (z - mu) / jnp.sqrt(var + 1e-5) * g + b


def _lstm_body(neigh_ref, x_ref, a0_ref, a1_ref, b0_ref, b1_ref, p_ref, bc_ref,
               g1_ref, bb1_ref, g3_ref, bb3_ref, out_ref, h0, c0, h1, c1,
               *, deg, d):
    t = pl.program_id(1)

    @pl.when(t == 0)
    def _():
        h0[...] = jnp.zeros_like(h0)
        c0[...] = jnp.zeros_like(c0)
        h1[...] = jnp.zeros_like(h1)
        c1[...] = jnp.zeros_like(c1)

    for r, (h, c, a_ref, b_ref) in enumerate(
        ((h0, c0, a0_ref, b0_ref), (h1, c1, a1_ref, b1_ref))
    ):
        adt = a_ref.dtype
        xt = neigh_ref[r, 0]
        xh = jnp.concatenate([xt.astype(adt), h[...].astype(adt)], axis=1)
        gates = jnp.dot(xh, a_ref[...], preferred_element_type=jnp.float32)
        gates = gates + b_ref[...]
        ii = gates[:, 0 * d:1 * d]
        ff = gates[:, 1 * d:2 * d]
        gg = gates[:, 2 * d:3 * d]
        oo = gates[:, 3 * d:4 * d]
        cn = jax.nn.sigmoid(ff) * c[...] + jax.nn.sigmoid(ii) * jnp.tanh(gg)
        c[...] = cn
        h[...] = jax.nn.sigmoid(oo) * jnp.tanh(cn)

    @pl.when(t == deg - 1)
    def _():
        xb = x_ref[...]
        pdt = p_ref.dtype
        hcat = jnp.concatenate(
            [xb.astype(pdt), h0[...].astype(pdt), h1[...].astype(pdt)], axis=1)
        y = jnp.dot(hcat, p_ref[...], preferred_element_type=jnp.float32)
        y = y + bc_ref[...]
        y = _leaky(_layer_norm(y, g1_ref[...], bb1_ref[...]))
        y = xb + y
        y = _leaky(_layer_norm(y, g3_ref[...], bb3_ref[...]))
        out_ref[...] = y


def _tc_lstm(neigh, x, a0, a1, b0, b1, p, bc, g1, bb1, g3, bb3, bn):
    n, d = x.shape
    deg = neigh.shape[1]
    grid = (n // bn, deg)
    full = lambda i, t: (0, 0)
    return pl.pallas_call(
        functools.partial(_lstm_body, deg=deg, d=d),
        grid=grid,
        in_specs=[
            pl.BlockSpec((2, 1, bn, d), lambda i, t: (0, t, i, 0)),
            pl.BlockSpec((bn, d), lambda i, t: (i, 0)),
            pl.BlockSpec((2 * d, 4 * d), full),
            pl.BlockSpec((2 * d, 4 * d), full),
            pl.BlockSpec((1, 4 * d), full),
            pl.BlockSpec((1, 4 * d), full),
            pl.BlockSpec((3 * d, d), full),
            pl.BlockSpec((1, d), full),
            pl.BlockSpec((1, d), full),
            pl.BlockSpec((1, d), full),
            pl.BlockSpec((1, d), full),
            pl.BlockSpec((1, d), full),
        ],
        out_specs=pl.BlockSpec((bn, d), lambda i, t: (i, 0)),
        out_shape=jax.ShapeDtypeStruct((n, d), jnp.float32),
        scratch_shapes=[pltpu.VMEM((bn, d), jnp.float32) for _ in range(4)],
        compiler_params=pltpu.CompilerParams(
            dimension_semantics=("arbitrary", "arbitrary"),
        ),
    )(neigh, x, a0, a1, b0, b1, p, bc, g1, bb1, g3, bb3)


def kernel(x, src0, src1, W_ih0, W_hh0, b_ih0, b_hh0, Ws0, Wn0, bc0,
           W_ih1, W_hh1, b_ih1, b_hh1, Ws1, Wn1, bc1,
           ln1_g, ln1_b, ln3_g, ln3_b):
    n, d = x.shape
    deg = src0.shape[0] // n

    idx0 = src0.astype(jnp.int32).reshape(n, deg).T
    idx1 = src1.astype(jnp.int32).reshape(n, deg).T
    idx_all = jnp.concatenate([idx0.reshape(-1), idx1.reshape(-1)])

    neigh = _sc_gather(x, idx_all).reshape(2, deg, n, d)

    bf = jnp.bfloat16
    a0 = jnp.concatenate([W_ih0.T, W_hh0.T], axis=0).astype(bf)
    a1 = jnp.concatenate([W_ih1.T, W_hh1.T], axis=0).astype(bf)
    b0 = (b_ih0 + b_hh0).reshape(1, 4 * d)
    b1 = (b_ih1 + b_hh1).reshape(1, 4 * d)
    p = jnp.concatenate([(Ws0 + Ws1).T, Wn0.T, Wn1.T], axis=0).astype(bf)
    bc = (bc0 + bc1).reshape(1, d)

    bn = 1000
    return _tc_lstm(neigh, x, a0, a1, b0, b1, p, bc,
                    ln1_g.reshape(1, d), ln1_b.reshape(1, d),
                    ln3_g.reshape(1, d), ln3_b.reshape(1, d), bn)


# speedup vs baseline: 5.4626x; 1.1103x over previous
import functools

import jax
import jax.numpy as jnp
from jax import lax
from jax.experimental import pallas as pl
from jax.experimental.pallas import tpu as pltpu
from jax.experimental.pallas import tpu_sc as plsc

_SC_CORES = 2
_SC_SUBCORES = 16
_NW = _SC_CORES * _SC_SUBCORES


def _sc_gather(x, idx_all):
    tot = idx_all.shape[0]
    d = x.shape[1]
    rows_w = tot // _NW
    assert rows_w * _NW == tot
    chunk = 128
    nfull = rows_w // chunk
    tail = rows_w - nfull * chunk
    chunks = [(j * chunk, chunk) for j in range(nfull)]
    if tail:
        chunks.append((nfull * chunk, tail))
    nch = len(chunks)

    mesh = plsc.VectorSubcoreMesh(core_axis_name="c", subcore_axis_name="s")

    @functools.partial(
        pl.kernel,
        out_type=jax.ShapeDtypeStruct((tot, d), x.dtype),
        mesh=mesh,
        scratch_types=[
            pltpu.VMEM((rows_w,), jnp.int32),
            pltpu.VMEM((chunk, d), x.dtype),
            pltpu.VMEM((chunk, d), x.dtype),
            pltpu.SemaphoreType.DMA,
            pltpu.SemaphoreType.DMA,
            pltpu.SemaphoreType.DMA,
            pltpu.SemaphoreType.DMA,
        ],
    )
    def gk(x_hbm, idx_hbm, out_hbm, idx_v, buf0, buf1, g0, g1, s0, s1):
        wid = lax.axis_index("s") * _SC_CORES + lax.axis_index("c")
        base = wid * rows_w
        pltpu.sync_copy(idx_hbm.at[pl.ds(base, rows_w)], idx_v)
        bufs = (buf0, buf1)
        gsems = (g0, g1)
        ssems = (s0, s1)
        gh = [None] * nch
        sh = [None] * nch
        off0, c0 = chunks[0]
        gh[0] = pltpu.async_copy(
            x_hbm.at[idx_v.at[pl.ds(off0, c0)]], bufs[0].at[pl.ds(0, c0)], gsems[0]
        )
        for j in range(nch):
            b = j & 1
            gh[j].wait()
            if j + 1 < nch:
                if j >= 1:
                    sh[j - 1].wait()
                offn, cn = chunks[j + 1]
                gh[j + 1] = pltpu.async_copy(
                    x_hbm.at[idx_v.at[pl.ds(offn, cn)]],
                    bufs[1 - b].at[pl.ds(0, cn)],
                    gsems[1 - b],
                )
            offj, cj = chunks[j]
            sh[j] = pltpu.async_copy(
                bufs[b].at[pl.ds(0, cj)],
                out_hbm.at[pl.ds(base + offj, cj)],
                ssems[b],
            )
        sh[nch - 1].wait()
        if nch >= 2:
            sh[nch - 2].wait()

    return gk(x, idx_all)


def _leaky(z):
    return jnp.where(z >= 0, z, 0.01 * z)


def _layer_norm(z, g, b):
    mu = jnp.mean(z, axis=-1, keepdims=True)
    var = jnp.mean((z - mu) * (z - mu), axis=-1, keepdims=True)
    return (z - mu) / jnp.sqrt(var + 1e-5) * g + b


def _lstm_body(neigh_ref, x_ref, a0_ref, a1_ref, b0_ref, b1_ref, p_ref, bc_ref,
               g1_ref, bb1_ref, g3_ref, bb3_ref, out_ref, h0, c0, h1, c1,
               *, deg, d):
    t = pl.program_id(1)

    @pl.when(t == 0)
    def _():
        h0[...] = jnp.zeros_like(h0)
        c0[...] = jnp.zeros_like(c0)
        h1[...] = jnp.zeros_like(h1)
        c1[...] = jnp.zeros_like(c1)

    for r, (h, c, a_ref, b_ref) in enumerate(
        ((h0, c0, a0_ref, b0_ref), (h1, c1, a1_ref, b1_ref))
    ):
        adt = a_ref.dtype
        xt = neigh_ref[r, 0]
        xh = jnp.concatenate([xt.astype(adt), h[...].astype(adt)], axis=1)
        gates = jnp.dot(xh, a_ref[...], preferred_element_type=jnp.float32)
        gates = gates + b_ref[...]
        ii = gates[:, 0 * d:1 * d]
        ff = gates[:, 1 * d:2 * d]
        gg = gates[:, 2 * d:3 * d]
        oo = gates[:, 3 * d:4 * d]
        si = 0.5 * jnp.tanh(ii) + 0.5
        sf = 0.5 * jnp.tanh(ff) + 0.5
        so = 0.5 * jnp.tanh(oo) + 0.5
        cn = sf * c[...] + si * jnp.tanh(gg)
        c[...] = cn
        h[...] = so * jnp.tanh(cn)

    @pl.when(t == deg - 1)
    def _():
        xb = x_ref[...]
        pdt = p_ref.dtype
        hcat = jnp.concatenate(
            [xb.astype(pdt), h0[...].astype(pdt), h1[...].astype(pdt)], axis=1)
        y = jnp.dot(hcat, p_ref[...], preferred_element_type=jnp.float32)
        y = y + bc_ref[...]
        y = _leaky(_layer_norm(y, g1_ref[...], bb1_ref[...]))
        y = xb + y
        y = _leaky(_layer_norm(y, g3_ref[...], bb3_ref[...]))
        out_ref[...] = y


def _tc_lstm(neigh, x, a0, a1, b0, b1, p, bc, g1, bb1, g3, bb3, bn):
    n, d = x.shape
    deg = neigh.shape[1]
    grid = (n // bn, deg)
    full = lambda i, t: (0, 0)
    return pl.pallas_call(
        functools.partial(_lstm_body, deg=deg, d=d),
        grid=grid,
        in_specs=[
            pl.BlockSpec((2, 1, bn, d), lambda i, t: (0, t, i, 0)),
            pl.BlockSpec((bn, d), lambda i, t: (i, 0)),
            pl.BlockSpec((2 * d, 4 * d), full),
            pl.BlockSpec((2 * d, 4 * d), full),
            pl.BlockSpec((1, 4 * d), full),
            pl.BlockSpec((1, 4 * d), full),
            pl.BlockSpec((3 * d, d), full),
            pl.BlockSpec((1, d), full),
            pl.BlockSpec((1, d), full),
            pl.BlockSpec((1, d), full),
            pl.BlockSpec((1, d), full),
            pl.BlockSpec((1, d), full),
        ],
        out_specs=pl.BlockSpec((bn, d), lambda i, t: (i, 0)),
        out_shape=jax.ShapeDtypeStruct((n, d), jnp.float32),
        scratch_shapes=[pltpu.VMEM((bn, d), jnp.float32) for _ in range(4)],
        compiler_params=pltpu.CompilerParams(
            dimension_semantics=("arbitrary", "arbitrary"),
        ),
    )(neigh, x, a0, a1, b0, b1, p, bc, g1, bb1, g3, bb3)


def kernel(x, src0, src1, W_ih0, W_hh0, b_ih0, b_hh0, Ws0, Wn0, bc0,
           W_ih1, W_hh1, b_ih1, b_hh1, Ws1, Wn1, bc1,
           ln1_g, ln1_b, ln3_g, ln3_b):
    n, d = x.shape
    deg = src0.shape[0] // n

    idx0 = src0.astype(jnp.int32).reshape(n, deg).T
    idx1 = src1.astype(jnp.int32).reshape(n, deg).T
    idx_all = jnp.concatenate([idx0.reshape(-1), idx1.reshape(-1)])

    neigh = _sc_gather(x, idx_all).reshape(2, deg, n, d)

    bf = jnp.bfloat16
    gate_scale = jnp.concatenate(
        [jnp.full((2 * d,), 0.5), jnp.ones((d,)), jnp.full((d,), 0.5)]
    ).astype(jnp.float32)
    a0 = (jnp.concatenate([W_ih0.T, W_hh0.T], axis=0) * gate_scale).astype(bf)
    a1 = (jnp.concatenate([W_ih1.T, W_hh1.T], axis=0) * gate_scale).astype(bf)
    b0 = ((b_ih0 + b_hh0) * gate_scale).reshape(1, 4 * d)
    b1 = ((b_ih1 + b_hh1) * gate_scale).reshape(1, 4 * d)
    p = jnp.concatenate([(Ws0 + Ws1).T, Wn0.T, Wn1.T], axis=0).astype(bf)
    bc = (bc0 + bc1).reshape(1, d)

    bn = 1000
    return _tc_lstm(neigh, x, a0, a1, b0, b1, p, bc,
                    ln1_g.reshape(1, d), ln1_b.reshape(1, d),
                    ln3_g.reshape(1, d), ln3_b.reshape(1, d), bn)
